# baseline (speedup 1.0000x reference)
import functools

import jax
import jax.numpy as jnp
from jax.experimental import pallas as pl
from jax.experimental.pallas import tpu as pltpu

B, D, H, C, E, EH, EO, T = 8192, 2048, 512, 256, 16, 256, 128, 3
TILE = 512


def _gelu(v):
    return 0.5 * v * (1.0 + jax.lax.erf(v * (2.0 ** -0.5)))


def _trunk_kernel(x_ref, task_ref, w1_ref, b1_ref, lng_ref, lnb_ref,
                  w2_ref, b2_ref, ew1_ref, eb1_ref, ew2_ref, eb2_ref,
                  gw1z_ref, gw1t_ref, gb1_ref, gw2_ref, gb2_ref,
                  out_ref, z_ref, gl_ref, gp_ref, load_ref, eo_ref,
                  ti_ref, tw_ref):
    i = pl.program_id(0)

    h = _gelu(jnp.dot(x_ref[...], w1_ref[...]) + b1_ref[...])
    mu = jnp.mean(h, axis=-1, keepdims=True)
    hc = h - mu
    var = jnp.mean(hc * hc, axis=-1, keepdims=True)
    hn = hc * jax.lax.rsqrt(var + 1e-5) * lng_ref[...] + lnb_ref[...]
    z = jnp.dot(hn, w2_ref[...]) + b2_ref[...]
    z_ref[...] = z

    g1 = _gelu(jnp.dot(z, gw1z_ref[...]) + jnp.dot(task_ref[...], gw1t_ref[...])
               + gb1_ref[...])
    gl = jnp.dot(g1, gw2_ref[...]) + gb2_ref[...]
    gl_ref[...] = gl

    m = jnp.max(gl, axis=-1, keepdims=True)
    eg = jnp.exp(gl - m)
    gp = eg / jnp.sum(eg, axis=-1, keepdims=True)
    gp_ref[...] = gp

    @pl.when(i == 0)
    def _():
        load_ref[...] = jnp.zeros_like(load_ref)
    load_ref[...] += jnp.sum(gp, axis=0, keepdims=True)

    iota = jax.lax.broadcasted_iota(jnp.int32, gl.shape, 1)
    m1 = jnp.max(gl, axis=-1, keepdims=True)
    i1 = jnp.min(jnp.where(gl == m1, iota, E), axis=-1, keepdims=True)
    masked = jnp.where(iota == i1, -jnp.inf, gl)
    m2 = jnp.max(masked, axis=-1, keepdims=True)
    i2 = jnp.min(jnp.where(masked == m2, iota, E), axis=-1, keepdims=True)
    ti_ref[...] = jnp.concatenate([i1, i2], axis=-1)
    e2 = jnp.exp(m2 - m1)
    w1 = 1.0 / (1.0 + e2)
    w2 = e2 * w1
    tw_ref[...] = jnp.concatenate([w1, w2], axis=-1)
    comb = (jnp.where(iota == i1, w1, 0.0) + jnp.where(iota == i2, w2, 0.0))

    zb = z.astype(jnp.bfloat16)
    eh_all = _gelu(
        jnp.dot(zb, ew1_ref[...], preferred_element_type=jnp.float32)
        + eb1_ref[...]).astype(jnp.bfloat16)
    acc = jnp.zeros((TILE, EO), jnp.float32)
    for e in range(E):
        eo = (jnp.dot(eh_all[:, e * EH:(e + 1) * EH], ew2_ref[e],
                      preferred_element_type=jnp.float32)
              + eb2_ref[:, e * EO:(e + 1) * EO])
        eo_ref[:, e, :] = eo
        acc += comb[:, e:e + 1] * eo
    out_ref[...] = acc


@functools.partial(jax.jit, static_argnums=())
def kernel(x, task_id, proj_w1, proj_b1, ln_g, ln_b, proj_w2, proj_b2,
           exp_w1, exp_b1, exp_w2, exp_b2, gate_w1, gate_b1, gate_w2, gate_b2):
    nsteps = B // TILE
    ew1 = jnp.transpose(exp_w1, (1, 0, 2)).reshape(C, E * EH).astype(jnp.bfloat16)
    ew2b = exp_w2.astype(jnp.bfloat16)
    eb1 = exp_b1.reshape(1, E * EH)
    eb2 = exp_b2.reshape(1, E * EO)
    gw1z = gate_w1[:C]
    gw1t = gate_w1[C:]

    full = lambda shape: pl.BlockSpec(shape, lambda i: tuple(0 for _ in shape))
    row = lambda shape: pl.BlockSpec(shape, lambda i: (i,) + (0,) * (len(shape) - 1))

    outs = (
        jax.ShapeDtypeStruct((B, EO), jnp.float32),
        jax.ShapeDtypeStruct((B, C), jnp.float32),
        jax.ShapeDtypeStruct((B, E), jnp.float32),
        jax.ShapeDtypeStruct((B, E), jnp.float32),
        jax.ShapeDtypeStruct((1, E), jnp.float32),
        jax.ShapeDtypeStruct((B, E, EO), jnp.float32),
        jax.ShapeDtypeStruct((B, 2), jnp.int32),
        jax.ShapeDtypeStruct((B, 2), jnp.float32),
    )
    out_specs = (
        row((TILE, EO)),
        row((TILE, C)),
        row((TILE, E)),
        row((TILE, E)),
        full((1, E)),
        row((TILE, E, EO)),
        row((TILE, 2)),
        row((TILE, 2)),
    )
    in_specs = [
        row((TILE, D)),
        row((TILE, T)),
        full((D, H)),
        full((1, H)),
        full((1, H)),
        full((1, H)),
        full((H, C)),
        full((1, C)),
        full((C, E * EH)),
        full((1, E * EH)),
        full((E, EH, EO)),
        full((1, E * EO)),
        full((C, 2 * E)),
        full((T, 2 * E)),
        full((1, 2 * E)),
        full((2 * E, E)),
        full((1, E)),
    ]

    (out, z, gl, gp, load_sums, eo, ti, tw) = pl.pallas_call(
        _trunk_kernel,
        grid=(nsteps,),
        in_specs=in_specs,
        out_specs=out_specs,
        out_shape=outs,
        compiler_params=pltpu.CompilerParams(
            dimension_semantics=("arbitrary",),
        ),
    )(x, task_id, proj_w1, proj_b1.reshape(1, H), ln_g.reshape(1, H),
      ln_b.reshape(1, H), proj_w2, proj_b2.reshape(1, C), ew1, eb1, ew2b,
      eb2, gw1z, gw1t, gate_b1.reshape(1, 2 * E), gate_w2,
      gate_b2.reshape(1, E))

    load = load_sums[0] / B
    lbl = 0.01 * (jnp.var(load, ddof=1) * E)
    return (out, z, gl, gp, lbl, eo, ti, tw)


# speedup vs baseline: 1.1016x; 1.1016x over previous
import functools

import jax
import jax.numpy as jnp
from jax.experimental import pallas as pl
from jax.experimental.pallas import tpu as pltpu

B, D, H, C, E, EH, EO, T = 8192, 2048, 512, 256, 16, 256, 128, 3
TILE = 512


def _gelu(v):
    return 0.5 * v * (1.0 + jax.lax.erf(v * (2.0 ** -0.5)))


def _trunk_kernel(x_ref, task_ref, w1_ref, b1_ref, lng_ref, lnb_ref,
                  w2_ref, b2_ref, ew1_ref, eb1_ref, ew2_ref, eb2_ref,
                  gw1z_ref, gw1t_ref, gb1_ref, gw2_ref, gb2_ref,
                  out_ref, z_ref, gl_ref, gp_ref, load_ref, eo_ref,
                  ti_ref, tw_ref):
    h = _gelu(jnp.dot(x_ref[...], w1_ref[...]) + b1_ref[...])
    mu = jnp.mean(h, axis=-1, keepdims=True)
    hc = h - mu
    var = jnp.mean(hc * hc, axis=-1, keepdims=True)
    hn = hc * jax.lax.rsqrt(var + 1e-5) * lng_ref[...] + lnb_ref[...]
    z = jnp.dot(hn, w2_ref[...]) + b2_ref[...]
    z_ref[...] = z

    g1 = _gelu(jnp.dot(z, gw1z_ref[...]) + jnp.dot(task_ref[...], gw1t_ref[...])
               + gb1_ref[...])
    gl = jnp.dot(g1, gw2_ref[...]) + gb2_ref[...]
    gl_ref[...] = gl

    m = jnp.max(gl, axis=-1, keepdims=True)
    eg = jnp.exp(gl - m)
    gp = eg / jnp.sum(eg, axis=-1, keepdims=True)
    gp_ref[...] = gp
    load_ref[...] = jnp.sum(gp, axis=0, keepdims=True)[None]

    iota = jax.lax.broadcasted_iota(jnp.int32, gl.shape, 1)
    m1 = jnp.max(gl, axis=-1, keepdims=True)
    i1 = jnp.min(jnp.where(gl == m1, iota, E), axis=-1, keepdims=True)
    masked = jnp.where(iota == i1, -jnp.inf, gl)
    m2 = jnp.max(masked, axis=-1, keepdims=True)
    i2 = jnp.min(jnp.where(masked == m2, iota, E), axis=-1, keepdims=True)
    ti_ref[...] = jnp.concatenate([i1, i2], axis=-1)
    e2 = jnp.exp(m2 - m1)
    w1 = 1.0 / (1.0 + e2)
    w2 = e2 * w1
    tw_ref[...] = jnp.concatenate([w1, w2], axis=-1)
    comb = (jnp.where(iota == i1, w1, 0.0) + jnp.where(iota == i2, w2, 0.0))

    acc = jnp.zeros((TILE, EO), jnp.float32)
    for e in range(E):
        eh = _gelu(jnp.dot(z, ew1_ref[e]) + eb1_ref[:, e * EH:(e + 1) * EH])
        eo = (jnp.dot(eh, ew2_ref[e]) + eb2_ref[:, e * EO:(e + 1) * EO])
        eo_ref[:, e, :] = eo
        acc += comb[:, e:e + 1] * eo
    out_ref[...] = acc


@functools.partial(jax.jit, static_argnums=())
def kernel(x, task_id, proj_w1, proj_b1, ln_g, ln_b, proj_w2, proj_b2,
           exp_w1, exp_b1, exp_w2, exp_b2, gate_w1, gate_b1, gate_w2, gate_b2):
    nsteps = B // TILE
    eb1 = exp_b1.reshape(1, E * EH)
    eb2 = exp_b2.reshape(1, E * EO)
    gw1z = gate_w1[:C]
    gw1t = gate_w1[C:]

    full = lambda shape: pl.BlockSpec(shape, lambda i: tuple(0 for _ in shape))
    row = lambda shape: pl.BlockSpec(shape, lambda i: (i,) + (0,) * (len(shape) - 1))

    outs = (
        jax.ShapeDtypeStruct((B, EO), jnp.float32),
        jax.ShapeDtypeStruct((B, C), jnp.float32),
        jax.ShapeDtypeStruct((B, E), jnp.float32),
        jax.ShapeDtypeStruct((B, E), jnp.float32),
        jax.ShapeDtypeStruct((nsteps, 1, E), jnp.float32),
        jax.ShapeDtypeStruct((B, E, EO), jnp.float32),
        jax.ShapeDtypeStruct((B, 2), jnp.int32),
        jax.ShapeDtypeStruct((B, 2), jnp.float32),
    )
    out_specs = (
        row((TILE, EO)),
        row((TILE, C)),
        row((TILE, E)),
        row((TILE, E)),
        row((1, 1, E)),
        row((TILE, E, EO)),
        row((TILE, 2)),
        row((TILE, 2)),
    )
    in_specs = [
        row((TILE, D)),
        row((TILE, T)),
        full((D, H)),
        full((1, H)),
        full((1, H)),
        full((1, H)),
        full((H, C)),
        full((1, C)),
        full((E, C, EH)),
        full((1, E * EH)),
        full((E, EH, EO)),
        full((1, E * EO)),
        full((C, 2 * E)),
        full((T, 2 * E)),
        full((1, 2 * E)),
        full((2 * E, E)),
        full((1, E)),
    ]

    (out, z, gl, gp, load_sums, eo, ti, tw) = pl.pallas_call(
        _trunk_kernel,
        grid=(nsteps,),
        in_specs=in_specs,
        out_specs=out_specs,
        out_shape=outs,
        compiler_params=pltpu.CompilerParams(
            dimension_semantics=("parallel",),
        ),
    )(x, task_id, proj_w1, proj_b1.reshape(1, H), ln_g.reshape(1, H),
      ln_b.reshape(1, H), proj_w2, proj_b2.reshape(1, C), exp_w1, eb1, exp_w2,
      eb2, gw1z, gw1t, gate_b1.reshape(1, 2 * E), gate_w2,
      gate_b2.reshape(1, E))

    load = jnp.sum(load_sums, axis=(0, 1)) / B
    lbl = 0.01 * (jnp.var(load, ddof=1) * E)
    return (out, z, gl, gp, lbl, eo, ti, tw)


# speedup vs baseline: 1.1032x; 1.0014x over previous
import functools

import jax
import jax.numpy as jnp
from jax.experimental import pallas as pl
from jax.experimental.pallas import tpu as pltpu

B, D, H, C, E, EH, EO, T = 8192, 2048, 512, 256, 16, 256, 128, 3
TILE = 512


def _gelu(v):
    return 0.5 * v * (1.0 + jax.lax.erf(v * (2.0 ** -0.5)))


def _trunk_kernel(x_ref, task_ref, w1_ref, b1_ref, lng_ref, lnb_ref,
                  w2_ref, b2_ref, ew1_ref, eb1_ref, ew2_ref, eb2_ref,
                  gw1z_ref, gw1t_ref, gb1_ref, gw2_ref, gb2_ref,
                  out_ref, z_ref, gl_ref, gp_ref, load_ref, eo_ref,
                  ti_ref, tw_ref):
    h = _gelu(jnp.dot(x_ref[...], w1_ref[...]) + b1_ref[...])
    mu = jnp.mean(h, axis=-1, keepdims=True)
    hc = h - mu
    var = jnp.mean(hc * hc, axis=-1, keepdims=True)
    hn = hc * jax.lax.rsqrt(var + 1e-5) * lng_ref[...] + lnb_ref[...]
    z = jnp.dot(hn, w2_ref[...]) + b2_ref[...]
    z_ref[...] = z

    g1 = _gelu(jnp.dot(z, gw1z_ref[...]) + jnp.dot(task_ref[...], gw1t_ref[...])
               + gb1_ref[...])
    gl = jnp.dot(g1, gw2_ref[...]) + gb2_ref[...]
    gl_ref[...] = gl

    m = jnp.max(gl, axis=-1, keepdims=True)
    eg = jnp.exp(gl - m)
    gp = eg / jnp.sum(eg, axis=-1, keepdims=True)
    gp_ref[...] = gp
    load_ref[...] = jnp.sum(gp, axis=0, keepdims=True)[None]

    iota = jax.lax.broadcasted_iota(jnp.int32, gl.shape, 1)
    m1 = jnp.max(gl, axis=-1, keepdims=True)
    i1 = jnp.min(jnp.where(gl == m1, iota, E), axis=-1, keepdims=True)
    masked = jnp.where(iota == i1, -jnp.inf, gl)
    m2 = jnp.max(masked, axis=-1, keepdims=True)
    i2 = jnp.min(jnp.where(masked == m2, iota, E), axis=-1, keepdims=True)
    ti_ref[...] = jnp.concatenate([i1, i2], axis=-1)
    e2 = jnp.exp(m2 - m1)
    w1 = 1.0 / (1.0 + e2)
    w2 = e2 * w1
    tw_ref[...] = jnp.concatenate([w1, w2], axis=-1)
    comb = (jnp.where(iota == i1, w1, 0.0) + jnp.where(iota == i2, w2, 0.0))

    zb = z.astype(jnp.bfloat16)
    acc = jnp.zeros((TILE, EO), jnp.float32)
    for e in range(E):
        ehf = (jnp.dot(zb, ew1_ref[e], preferred_element_type=jnp.float32)
               + eb1_ref[:, e * EH:(e + 1) * EH])
        ehb = _gelu(ehf.astype(jnp.bfloat16))
        eo = (jnp.dot(ehb, ew2_ref[e], preferred_element_type=jnp.float32)
              + eb2_ref[:, e * EO:(e + 1) * EO])
        eo_ref[:, e, :] = eo
        acc += comb[:, e:e + 1] * eo
    out_ref[...] = acc


@functools.partial(jax.jit, static_argnums=())
def kernel(x, task_id, proj_w1, proj_b1, ln_g, ln_b, proj_w2, proj_b2,
           exp_w1, exp_b1, exp_w2, exp_b2, gate_w1, gate_b1, gate_w2, gate_b2):
    nsteps = B // TILE
    ew1b = exp_w1.astype(jnp.bfloat16)
    ew2b = exp_w2.astype(jnp.bfloat16)
    eb1 = exp_b1.reshape(1, E * EH)
    eb2 = exp_b2.reshape(1, E * EO)
    gw1z = gate_w1[:C]
    gw1t = gate_w1[C:]

    full = lambda shape: pl.BlockSpec(shape, lambda i: tuple(0 for _ in shape))
    row = lambda shape: pl.BlockSpec(shape, lambda i: (i,) + (0,) * (len(shape) - 1))

    outs = (
        jax.ShapeDtypeStruct((B, EO), jnp.float32),
        jax.ShapeDtypeStruct((B, C), jnp.float32),
        jax.ShapeDtypeStruct((B, E), jnp.float32),
        jax.ShapeDtypeStruct((B, E), jnp.float32),
        jax.ShapeDtypeStruct((nsteps, 1, E), jnp.float32),
        jax.ShapeDtypeStruct((B, E, EO), jnp.float32),
        jax.ShapeDtypeStruct((B, 2), jnp.int32),
        jax.ShapeDtypeStruct((B, 2), jnp.float32),
    )
    out_specs = (
        row((TILE, EO)),
        row((TILE, C)),
        row((TILE, E)),
        row((TILE, E)),
        row((1, 1, E)),
        row((TILE, E, EO)),
        row((TILE, 2)),
        row((TILE, 2)),
    )
    in_specs = [
        row((TILE, D)),
        row((TILE, T)),
        full((D, H)),
        full((1, H)),
        full((1, H)),
        full((1, H)),
        full((H, C)),
        full((1, C)),
        full((E, C, EH)),
        full((1, E * EH)),
        full((E, EH, EO)),
        full((1, E * EO)),
        full((C, 2 * E)),
        full((T, 2 * E)),
        full((1, 2 * E)),
        full((2 * E, E)),
        full((1, E)),
    ]

    (out, z, gl, gp, load_sums, eo, ti, tw) = pl.pallas_call(
        _trunk_kernel,
        grid=(nsteps,),
        in_specs=in_specs,
        out_specs=out_specs,
        out_shape=outs,
        compiler_params=pltpu.CompilerParams(
            dimension_semantics=("parallel",),
        ),
    )(x, task_id, proj_w1, proj_b1.reshape(1, H), ln_g.reshape(1, H),
      ln_b.reshape(1, H), proj_w2, proj_b2.reshape(1, C), ew1b, eb1, ew2b,
      eb2, gw1z, gw1t, gate_b1.reshape(1, 2 * E), gate_w2,
      gate_b2.reshape(1, E))

    load = jnp.sum(load_sums, axis=(0, 1)) / B
    lbl = 0.01 * (jnp.var(load, ddof=1) * E)
    return (out, z, gl, gp, lbl, eo, ti, tw)


# speedup vs baseline: 1.1082x; 1.0046x over previous
import functools

import jax
import jax.numpy as jnp
from jax.experimental import pallas as pl
from jax.experimental.pallas import tpu as pltpu

B, D, H, C, E, EH, EO, T = 8192, 2048, 512, 256, 16, 256, 128, 3
TILE = 1024


def _gelu(v):
    return 0.5 * v * (1.0 + jax.lax.erf(v * (2.0 ** -0.5)))


def _trunk_kernel(x_ref, task_ref, w1_ref, b1_ref, lng_ref, lnb_ref,
                  w2_ref, b2_ref, ew1_ref, eb1_ref, ew2_ref, eb2_ref,
                  gw1z_ref, gw1t_ref, gb1_ref, gw2_ref, gb2_ref,
                  out_ref, z_ref, gl_ref, gp_ref, load_ref, eo_ref,
                  ti_ref, tw_ref):
    h = _gelu(jnp.dot(x_ref[...], w1_ref[...]) + b1_ref[...])
    mu = jnp.mean(h, axis=-1, keepdims=True)
    hc = h - mu
    var = jnp.mean(hc * hc, axis=-1, keepdims=True)
    hn = hc * jax.lax.rsqrt(var + 1e-5) * lng_ref[...] + lnb_ref[...]
    z = jnp.dot(hn, w2_ref[...]) + b2_ref[...]
    z_ref[...] = z

    g1 = _gelu(jnp.dot(z, gw1z_ref[...]) + jnp.dot(task_ref[...], gw1t_ref[...])
               + gb1_ref[...])
    gl = jnp.dot(g1, gw2_ref[...]) + gb2_ref[...]
    gl_ref[...] = gl

    m = jnp.max(gl, axis=-1, keepdims=True)
    eg = jnp.exp(gl - m)
    gp = eg / jnp.sum(eg, axis=-1, keepdims=True)
    gp_ref[...] = gp
    load_ref[...] = jnp.sum(gp, axis=0, keepdims=True)[None]

    iota = jax.lax.broadcasted_iota(jnp.int32, gl.shape, 1)
    m1 = jnp.max(gl, axis=-1, keepdims=True)
    i1 = jnp.min(jnp.where(gl == m1, iota, E), axis=-1, keepdims=True)
    masked = jnp.where(iota == i1, -jnp.inf, gl)
    m2 = jnp.max(masked, axis=-1, keepdims=True)
    i2 = jnp.min(jnp.where(masked == m2, iota, E), axis=-1, keepdims=True)
    ti_ref[...] = jnp.concatenate([i1, i2], axis=-1)
    e2 = jnp.exp(m2 - m1)
    w1 = 1.0 / (1.0 + e2)
    w2 = e2 * w1
    tw_ref[...] = jnp.concatenate([w1, w2], axis=-1)
    comb = (jnp.where(iota == i1, w1, 0.0) + jnp.where(iota == i2, w2, 0.0))

    zb = z.astype(jnp.bfloat16)
    acc = jnp.zeros((TILE, EO), jnp.float32)
    for e in range(E):
        ehf = (jnp.dot(zb, ew1_ref[e], preferred_element_type=jnp.float32)
               + eb1_ref[:, e * EH:(e + 1) * EH])
        ehb = _gelu(ehf.astype(jnp.bfloat16))
        eo = (jnp.dot(ehb, ew2_ref[e], preferred_element_type=jnp.float32)
              + eb2_ref[:, e * EO:(e + 1) * EO])
        eo_ref[:, e, :] = eo
        acc += comb[:, e:e + 1] * eo
    out_ref[...] = acc


@functools.partial(jax.jit, static_argnums=())
def kernel(x, task_id, proj_w1, proj_b1, ln_g, ln_b, proj_w2, proj_b2,
           exp_w1, exp_b1, exp_w2, exp_b2, gate_w1, gate_b1, gate_w2, gate_b2):
    nsteps = B // TILE
    ew1b = exp_w1.astype(jnp.bfloat16)
    ew2b = exp_w2.astype(jnp.bfloat16)
    eb1 = exp_b1.reshape(1, E * EH)
    eb2 = exp_b2.reshape(1, E * EO)
    gw1z = gate_w1[:C]
    gw1t = gate_w1[C:]

    full = lambda shape: pl.BlockSpec(shape, lambda i: tuple(0 for _ in shape))
    row = lambda shape: pl.BlockSpec(shape, lambda i: (i,) + (0,) * (len(shape) - 1))

    outs = (
        jax.ShapeDtypeStruct((B, EO), jnp.float32),
        jax.ShapeDtypeStruct((B, C), jnp.float32),
        jax.ShapeDtypeStruct((B, E), jnp.float32),
        jax.ShapeDtypeStruct((B, E), jnp.float32),
        jax.ShapeDtypeStruct((nsteps, 1, E), jnp.float32),
        jax.ShapeDtypeStruct((B, E, EO), jnp.float32),
        jax.ShapeDtypeStruct((B, 2), jnp.int32),
        jax.ShapeDtypeStruct((B, 2), jnp.float32),
    )
    out_specs = (
        row((TILE, EO)),
        row((TILE, C)),
        row((TILE, E)),
        row((TILE, E)),
        row((1, 1, E)),
        row((TILE, E, EO)),
        row((TILE, 2)),
        row((TILE, 2)),
    )
    in_specs = [
        row((TILE, D)),
        row((TILE, T)),
        full((D, H)),
        full((1, H)),
        full((1, H)),
        full((1, H)),
        full((H, C)),
        full((1, C)),
        full((E, C, EH)),
        full((1, E * EH)),
        full((E, EH, EO)),
        full((1, E * EO)),
        full((C, 2 * E)),
        full((T, 2 * E)),
        full((1, 2 * E)),
        full((2 * E, E)),
        full((1, E)),
    ]

    (out, z, gl, gp, load_sums, eo, ti, tw) = pl.pallas_call(
        _trunk_kernel,
        grid=(nsteps,),
        in_specs=in_specs,
        out_specs=out_specs,
        out_shape=outs,
        compiler_params=pltpu.CompilerParams(
            dimension_semantics=("parallel",),
        ),
    )(x, task_id, proj_w1, proj_b1.reshape(1, H), ln_g.reshape(1, H),
      ln_b.reshape(1, H), proj_w2, proj_b2.reshape(1, C), ew1b, eb1, ew2b,
      eb2, gw1z, gw1t, gate_b1.reshape(1, 2 * E), gate_w2,
      gate_b2.reshape(1, E))

    load = jnp.sum(load_sums, axis=(0, 1)) / B
    lbl = 0.01 * (jnp.var(load, ddof=1) * E)
    return (out, z, gl, gp, lbl, eo, ti, tw)
